# baseline (speedup 1.0000x reference)
import functools

import jax
import jax.numpy as jnp
from jax import lax
from jax.experimental import pallas as pl
from jax.experimental.pallas import tpu as pltpu
from jax.experimental.pallas import tpu_sc as plsc

_NC = 2
_NS = 16
_L = 16
_NW = _NC * _NS

_TABLE_PAD = 128
_NCH = 4
_NB = 2


def _lookup_add(xf, zi, table):
  n = xf.shape[0]
  per_w = n // _NW
  chunk = per_w // _NCH
  mesh = plsc.VectorSubcoreMesh(
      core_axis_name="c", subcore_axis_name="s", num_cores=_NC
  )

  @functools.partial(
      pl.kernel,
      out_type=jax.ShapeDtypeStruct((n,), jnp.float32),
      mesh=mesh,
      compiler_params=pltpu.CompilerParams(needs_layout_passes=False),
      scratch_types=[
          pltpu.VMEM((_TABLE_PAD,), jnp.float32),
          pltpu.VMEM((chunk,), jnp.int32),
          pltpu.VMEM((chunk,), jnp.int32),
          pltpu.VMEM((chunk,), jnp.float32),
          pltpu.VMEM((chunk,), jnp.float32),
          pltpu.SemaphoreType.DMA((_NB,)),
          pltpu.SemaphoreType.DMA((_NB,)),
          pltpu.SemaphoreType.DMA((_NB,)),
      ],
  )
  def body(x_hbm, z_hbm, tab_hbm, out_hbm, tab_v, z_v0, z_v1, o_v0, o_v1,
           sem_z, sem_x, sem_o):
    wid = lax.axis_index("s") * _NC + lax.axis_index("c")
    base = wid * per_w
    z_bufs = (z_v0, z_v1)
    o_bufs = (o_v0, o_v1)

    def start_in(k):
      b = k % _NB
      lo = base + k * chunk
      cz = pltpu.async_copy(z_hbm.at[pl.ds(lo, chunk)], z_bufs[b],
                            sem_z.at[b])
      cx = pltpu.async_copy(x_hbm.at[pl.ds(lo, chunk)], o_bufs[b],
                            sem_x.at[b])
      return cz, cx

    in_copies = {0: start_in(0)}
    out_copies = {}
    pltpu.sync_copy(tab_hbm, tab_v)

    for k in range(_NCH):
      b = k % _NB
      if k + 1 < _NCH:
        if k - 1 >= 0:
          out_copies[k - 1].wait()
        in_copies[k + 1] = start_in(k + 1)
      cz, cx = in_copies.pop(k)
      cz.wait()
      cx.wait()

      @plsc.parallel_loop(0, chunk, _L, unroll=8)
      def _(i, _zb=z_bufs[b], _ob=o_bufs[b]):
        vals = plsc.load_gather(tab_v, [_zb[pl.ds(i, _L)]])
        plsc.addupdate(_ob.at[pl.ds(i, _L)], vals)

      out_copies[k] = pltpu.async_copy(
          o_bufs[b], out_hbm.at[pl.ds(base + k * chunk, chunk)], sem_o.at[b]
      )

    for k in range(_NCH - 2, _NCH):
      out_copies[k].wait()

  return body(xf, zi, table)


def _tc_lookup_add(xr, zr, tab_row):
  rows, cols = xr.shape
  bm = min(512, rows)

  def body(tab_ref, x_ref, z_ref, o_ref):
    tab = jnp.broadcast_to(tab_ref[...], (bm, cols))
    vals = jnp.take_along_axis(tab, z_ref[...], axis=1)
    o_ref[...] = x_ref[...] + vals

  return pl.pallas_call(
      body,
      grid=(rows // bm,),
      in_specs=[
          pl.BlockSpec((1, cols), lambda i: (0, 0)),
          pl.BlockSpec((bm, cols), lambda i: (i, 0)),
          pl.BlockSpec((bm, cols), lambda i: (i, 0)),
      ],
      out_specs=pl.BlockSpec((bm, cols), lambda i: (i, 0)),
      out_shape=jax.ShapeDtypeStruct((rows, cols), jnp.float32),
  )(tab_row, xr, zr)


_SC_SIXTEENTHS = 8


@jax.jit
def _combined(xf, zi, table):
  n = xf.shape[0]
  n_sc = (n * _SC_SIXTEENTHS // 16) // (_NW * _NCH * _L) * (_NW * _NCH * _L)
  out_sc = _lookup_add(xf[:n_sc], zi[:n_sc], table)
  xr = xf[n_sc:].reshape(-1, 128)
  zr = zi[n_sc:].reshape(-1, 128)
  out_tc = _tc_lookup_add(xr, zr, table.reshape(1, 128)).reshape(-1)
  return jnp.concatenate([out_sc, out_tc])


def kernel(x, z, atomref_weight):
  table = jnp.pad(
      atomref_weight[:, 0], (0, _TABLE_PAD - atomref_weight.shape[0])
  )
  out = _combined(x[:, 0], z.astype(jnp.int32), table)
  return out[:, None]


# speedup vs baseline: 1.4306x; 1.4306x over previous
import functools

import jax
import jax.numpy as jnp
from jax import lax
from jax.experimental import pallas as pl
from jax.experimental.pallas import tpu as pltpu
from jax.experimental.pallas import tpu_sc as plsc

_NC = 2
_NS = 16
_L = 16
_NW = _NC * _NS

_TABLE_PAD = 128
_NCH = 2
_NB = 2


def _lookup_add(xf, zi, table):
  n = xf.shape[0]
  per_w = n // _NW
  chunk = per_w // _NCH
  mesh = plsc.VectorSubcoreMesh(
      core_axis_name="c", subcore_axis_name="s", num_cores=_NC
  )

  @functools.partial(
      pl.kernel,
      out_type=jax.ShapeDtypeStruct((n,), jnp.float32),
      mesh=mesh,
      compiler_params=pltpu.CompilerParams(needs_layout_passes=False),
      scratch_types=[
          pltpu.VMEM((_TABLE_PAD,), jnp.float32),
          pltpu.VMEM((chunk,), jnp.int32),
          pltpu.VMEM((chunk,), jnp.int32),
          pltpu.VMEM((chunk,), jnp.float32),
          pltpu.VMEM((chunk,), jnp.float32),
          pltpu.SemaphoreType.DMA((_NB,)),
          pltpu.SemaphoreType.DMA((_NB,)),
          pltpu.SemaphoreType.DMA((_NB,)),
      ],
  )
  def body(x_hbm, z_hbm, tab_hbm, out_hbm, tab_v, z_v0, z_v1, o_v0, o_v1,
           sem_z, sem_x, sem_o):
    wid = lax.axis_index("s") * _NC + lax.axis_index("c")
    base = wid * per_w
    z_bufs = (z_v0, z_v1)
    o_bufs = (o_v0, o_v1)

    def start_in(k):
      b = k % _NB
      lo = base + k * chunk
      cz = pltpu.async_copy(z_hbm.at[pl.ds(lo, chunk)], z_bufs[b],
                            sem_z.at[b])
      cx = pltpu.async_copy(x_hbm.at[pl.ds(lo, chunk)], o_bufs[b],
                            sem_x.at[b])
      return cz, cx

    in_copies = {0: start_in(0)}
    out_copies = {}
    pltpu.sync_copy(tab_hbm, tab_v)

    for k in range(_NCH):
      b = k % _NB
      if k + 1 < _NCH:
        if k - 1 >= 0:
          out_copies[k - 1].wait()
        in_copies[k + 1] = start_in(k + 1)
      cz, cx = in_copies.pop(k)
      cz.wait()
      cx.wait()

      @plsc.parallel_loop(0, chunk, _L, unroll=8)
      def _(i, _zb=z_bufs[b], _ob=o_bufs[b]):
        vals = plsc.load_gather(tab_v, [_zb[pl.ds(i, _L)]])
        plsc.addupdate(_ob.at[pl.ds(i, _L)], vals)

      out_copies[k] = pltpu.async_copy(
          o_bufs[b], out_hbm.at[pl.ds(base + k * chunk, chunk)], sem_o.at[b]
      )

    for k in range(_NCH - 2, _NCH):
      out_copies[k].wait()

  return body(xf, zi, table)


def _tc_lookup_add(xr, zr, tab_row):
  rows, cols = xr.shape
  bm = min(512, rows)

  def body(tab_ref, x_ref, z_ref, o_ref):
    tab = jnp.broadcast_to(tab_ref[...], (bm, cols))
    vals = jnp.take_along_axis(tab, z_ref[...], axis=1)
    o_ref[...] = x_ref[...] + vals

  return pl.pallas_call(
      body,
      grid=(rows // bm,),
      in_specs=[
          pl.BlockSpec((1, cols), lambda i: (0, 0)),
          pl.BlockSpec((bm, cols), lambda i: (i, 0)),
          pl.BlockSpec((bm, cols), lambda i: (i, 0)),
      ],
      out_specs=pl.BlockSpec((bm, cols), lambda i: (i, 0)),
      out_shape=jax.ShapeDtypeStruct((rows, cols), jnp.float32),
  )(tab_row, xr, zr)


@jax.jit
def _combined(xf, zi, table):
  return _lookup_add(xf, zi, table)


def kernel(x, z, atomref_weight):
  table = jnp.pad(
      atomref_weight[:, 0], (0, _TABLE_PAD - atomref_weight.shape[0])
  )
  out = _combined(x[:, 0], z.astype(jnp.int32), table)
  return out[:, None]


# speedup vs baseline: 1.4729x; 1.0296x over previous
import functools

import jax
import jax.numpy as jnp
from jax import lax
from jax.experimental import pallas as pl
from jax.experimental.pallas import tpu as pltpu
from jax.experimental.pallas import tpu_sc as plsc

_NC = 2
_NS = 16
_L = 16
_NW = _NC * _NS

_TABLE_PAD = 128
_NCH = 2
_NB = 2


def _lookup_add(xf, zi, table):
  n = xf.shape[0]
  per_w = n // _NW
  chunk = per_w // _NCH
  mesh = plsc.VectorSubcoreMesh(
      core_axis_name="c", subcore_axis_name="s", num_cores=_NC
  )

  @functools.partial(
      pl.kernel,
      out_type=jax.ShapeDtypeStruct((n,), jnp.float32),
      mesh=mesh,
      compiler_params=pltpu.CompilerParams(needs_layout_passes=False),
      scratch_types=[
          pltpu.VMEM((_TABLE_PAD,), jnp.float32),
          pltpu.VMEM((chunk,), jnp.int32),
          pltpu.VMEM((chunk,), jnp.int32),
          pltpu.VMEM((chunk,), jnp.float32),
          pltpu.VMEM((chunk,), jnp.float32),
          pltpu.SemaphoreType.DMA((_NB,)),
          pltpu.SemaphoreType.DMA((_NB,)),
          pltpu.SemaphoreType.DMA((_NB,)),
      ],
  )
  def body(x_hbm, z_hbm, tab_hbm, out_hbm, tab_v, z_v0, z_v1, o_v0, o_v1,
           sem_z, sem_x, sem_o):
    wid = lax.axis_index("s") * _NC + lax.axis_index("c")
    base = wid * per_w
    z_bufs = (z_v0, z_v1)
    o_bufs = (o_v0, o_v1)

    def start_in(k):
      b = k % _NB
      lo = base + k * chunk
      cz = pltpu.async_copy(z_hbm.at[pl.ds(lo, chunk)], z_bufs[b],
                            sem_z.at[b])
      cx = pltpu.async_copy(x_hbm.at[pl.ds(lo, chunk)], o_bufs[b],
                            sem_x.at[b])
      return cz, cx

    in_copies = {0: start_in(0)}
    out_copies = {}
    pltpu.sync_copy(tab_hbm, tab_v.at[pl.ds(0, tab_hbm.shape[0])])

    for k in range(_NCH):
      b = k % _NB
      if k + 1 < _NCH:
        if k - 1 >= 0:
          out_copies[k - 1].wait()
        in_copies[k + 1] = start_in(k + 1)
      cz, cx = in_copies.pop(k)
      cz.wait()
      cx.wait()

      @plsc.parallel_loop(0, chunk, _L, unroll=8)
      def _(i, _zb=z_bufs[b], _ob=o_bufs[b]):
        vals = plsc.load_gather(tab_v, [_zb[pl.ds(i, _L)]])
        plsc.addupdate(_ob.at[pl.ds(i, _L)], vals)

      out_copies[k] = pltpu.async_copy(
          o_bufs[b], out_hbm.at[pl.ds(base + k * chunk, chunk)], sem_o.at[b]
      )

    for k in range(_NCH - 2, _NCH):
      out_copies[k].wait()

  return body(xf, zi, table)


def _tc_lookup_add(xr, zr, tab_row):
  rows, cols = xr.shape
  bm = min(512, rows)

  def body(tab_ref, x_ref, z_ref, o_ref):
    tab = jnp.broadcast_to(tab_ref[...], (bm, cols))
    vals = jnp.take_along_axis(tab, z_ref[...], axis=1)
    o_ref[...] = x_ref[...] + vals

  return pl.pallas_call(
      body,
      grid=(rows // bm,),
      in_specs=[
          pl.BlockSpec((1, cols), lambda i: (0, 0)),
          pl.BlockSpec((bm, cols), lambda i: (i, 0)),
          pl.BlockSpec((bm, cols), lambda i: (i, 0)),
      ],
      out_specs=pl.BlockSpec((bm, cols), lambda i: (i, 0)),
      out_shape=jax.ShapeDtypeStruct((rows, cols), jnp.float32),
  )(tab_row, xr, zr)


@jax.jit
def _combined(xf, zi, table):
  return _lookup_add(xf, zi, table)


def kernel(x, z, atomref_weight):
  out = _combined(x[:, 0], z.astype(jnp.int32), atomref_weight[:, 0])
  return out[:, None]
